# Initial kernel scaffold; baseline (speedup 1.0000x reference)
import functools

import jax
import jax.numpy as jnp
from jax import lax
from jax.experimental import pallas as pl
from jax.experimental.pallas import tpu as pltpu
from jax.experimental.pallas import tpu_sc as plsc

_NC = 2
_NS = 16
_NW = _NC * _NS

_CHUNK = 128
_PAD_ROWS = 240
_BLK = 512
_NSEG = 64


def _build_degree_kernel(npad, cw):
    rows_per_tile = npad // _NS
    mesh = plsc.VectorSubcoreMesh(core_axis_name="c", subcore_axis_name="s")

    @functools.partial(
        pl.kernel,
        out_type=jax.ShapeDtypeStruct((_NC, _NS, rows_per_tile), jnp.float32),
        mesh=mesh,
        scratch_types=[
            pltpu.VMEM((cw, _CHUNK), jnp.int32),
            pltpu.VMEM((_CHUNK,), jnp.float32),
            pltpu.VMEM((rows_per_tile,), jnp.float32),
            pltpu.VMEM_SHARED((npad,), jnp.float32),
        ],
    )
    def deg_kernel(dst_hbm, out_hbm, idx_v, ones_v, zero_v, acc_sh):
        c = lax.axis_index("c")
        s = lax.axis_index("s")
        w = s * _NC + c
        row0 = s * rows_per_tile

        def _fill(i, carry):
            zero_v[pl.ds(i * 16, 16)] = jnp.zeros((16,), jnp.float32)
            return carry

        lax.fori_loop(0, rows_per_tile // 16, _fill, 0)

        def _fill1(i, carry):
            ones_v[pl.ds(i * 16, 16)] = jnp.ones((16,), jnp.float32)
            return carry

        lax.fori_loop(0, _CHUNK // 16, _fill1, 0)

        pltpu.sync_copy(zero_v, acc_sh.at[pl.ds(row0, rows_per_tile)])
        pltpu.sync_copy(dst_hbm.at[w], idx_v)
        plsc.subcore_barrier()

        def _scat(j, carry):
            pltpu.sync_copy(ones_v, acc_sh.at[idx_v.at[j]], add=True)
            return carry

        lax.fori_loop(0, cw, _scat, 0)
        plsc.subcore_barrier()
        pltpu.sync_copy(acc_sh.at[pl.ds(row0, rows_per_tile)], out_hbm.at[c, s])

    return deg_kernel


def _build_propagate_kernel(npad, d, cw):
    rows_per_tile = npad // _NS
    nzero = rows_per_tile // _CHUNK
    mesh = plsc.VectorSubcoreMesh(core_axis_name="c", subcore_axis_name="s")

    @functools.partial(
        pl.kernel,
        out_type=jax.ShapeDtypeStruct((_NC, _NS, rows_per_tile, d), jnp.float32),
        mesh=mesh,
        scratch_types=[
            pltpu.VMEM((cw, _CHUNK), jnp.int32),
            pltpu.VMEM((cw, _CHUNK), jnp.int32),
            pltpu.VMEM((2, _CHUNK, d), jnp.float32),
            pltpu.VMEM_SHARED((npad, d), jnp.float32),
            pltpu.SemaphoreType.DMA,
            pltpu.SemaphoreType.DMA,
        ],
    )
    def prop_kernel(h_hbm, src_hbm, dst_hbm, out_hbm,
                    sidx_v, didx_v, rows_v, acc_sh, sem0, sem1):
        c = lax.axis_index("c")
        s = lax.axis_index("s")
        w = s * _NC + c
        row0 = s * rows_per_tile

        def _zr(r, carry):
            for kk in range(d // 16):
                rows_v[0, r, pl.ds(kk * 16, 16)] = jnp.zeros((16,), jnp.float32)
            return carry

        lax.fori_loop(0, _CHUNK, _zr, 0)
        for k in range(nzero):
            pltpu.sync_copy(rows_v.at[0],
                            acc_sh.at[pl.ds(row0 + k * _CHUNK, _CHUNK)])
        pltpu.sync_copy(src_hbm.at[w], sidx_v)
        pltpu.sync_copy(dst_hbm.at[w], didx_v)
        plsc.subcore_barrier()

        pltpu.async_copy(h_hbm.at[sidx_v.at[0]], rows_v.at[0], sem0)

        def _pair(p, carry):
            j0 = 2 * p
            pltpu.make_async_copy(h_hbm.at[sidx_v.at[j0]],
                                  rows_v.at[0], sem0).wait()
            pltpu.async_copy(h_hbm.at[sidx_v.at[j0 + 1]], rows_v.at[1], sem1)
            pltpu.sync_copy(rows_v.at[0], acc_sh.at[didx_v.at[j0]], add=True)
            pltpu.make_async_copy(h_hbm.at[sidx_v.at[j0 + 1]],
                                  rows_v.at[1], sem1).wait()

            @pl.when(j0 + 2 < cw)
            def _():
                pltpu.async_copy(h_hbm.at[sidx_v.at[j0 + 2]],
                                 rows_v.at[0], sem0)

            pltpu.sync_copy(rows_v.at[1], acc_sh.at[didx_v.at[j0 + 1]],
                            add=True)
            return carry

        lax.fori_loop(0, cw // 2, _pair, 0)
        plsc.subcore_barrier()
        pltpu.sync_copy(acc_sh.at[pl.ds(row0, rows_per_tile)],
                        out_hbm.at[c, s])

    return prop_kernel


def _tc_scale_in(npad, d):
    grid = npad // _BLK

    def body(x_ref, w_ref, degp_ref, out_ref):
        deg = degp_ref[0] + degp_ref[1] + 1.0
        dinv = lax.rsqrt(deg)
        h = jnp.dot(x_ref[...], w_ref[...], preferred_element_type=jnp.float32)
        out_ref[...] = h * dinv[:, None]

    return pl.pallas_call(
        body,
        grid=(grid,),
        in_specs=[
            pl.BlockSpec((_BLK, d), lambda i: (i, 0)),
            pl.BlockSpec((d, d), lambda i: (0, 0)),
            pl.BlockSpec((_NC, _BLK), lambda i: (0, i)),
        ],
        out_specs=pl.BlockSpec((_BLK, d), lambda i: (i, 0)),
        out_shape=jax.ShapeDtypeStruct((npad, d), jnp.float32),
    )


def _tc_layer_mid(npad, d, n):
    grid = npad // _BLK

    def body(s_ref, hp_ref, degp_ref, b_ref, w_ref, out_ref):
        i = pl.program_id(0)
        deg = degp_ref[0] + degp_ref[1] + 1.0
        dinv = lax.rsqrt(deg)[:, None]
        tot = s_ref[0] + s_ref[1] + hp_ref[...]
        h1 = jnp.maximum(tot * dinv + b_ref[...], 0.0)
        out = jnp.dot(h1, w_ref[...], preferred_element_type=jnp.float32) * dinv
        rid = i * _BLK + lax.broadcasted_iota(jnp.int32, (_BLK, 1), 0)
        out_ref[...] = jnp.where(rid < n, out, 0.0)

    return pl.pallas_call(
        body,
        grid=(grid,),
        in_specs=[
            pl.BlockSpec((_NC, _BLK, d), lambda i: (0, i, 0)),
            pl.BlockSpec((_BLK, d), lambda i: (i, 0)),
            pl.BlockSpec((_NC, _BLK), lambda i: (0, i)),
            pl.BlockSpec((1, d), lambda i: (0, 0)),
            pl.BlockSpec((d, d), lambda i: (0, 0)),
        ],
        out_specs=pl.BlockSpec((_BLK, d), lambda i: (i, 0)),
        out_shape=jax.ShapeDtypeStruct((npad, d), jnp.float32),
    )


def _tc_layer_out(npad, d):
    grid = npad // _BLK

    def body(s_ref, hp_ref, degp_ref, b_ref, out_ref):
        deg = degp_ref[0] + degp_ref[1] + 1.0
        dinv = lax.rsqrt(deg)[:, None]
        tot = s_ref[0] + s_ref[1] + hp_ref[...]
        out_ref[...] = tot * dinv + b_ref[...]

    return pl.pallas_call(
        body,
        grid=(grid,),
        in_specs=[
            pl.BlockSpec((_NC, _BLK, d), lambda i: (0, i, 0)),
            pl.BlockSpec((_BLK, d), lambda i: (i, 0)),
            pl.BlockSpec((_NC, _BLK), lambda i: (0, i)),
            pl.BlockSpec((1, d), lambda i: (0, 0)),
        ],
        out_specs=pl.BlockSpec((_BLK, d), lambda i: (i, 0)),
        out_shape=jax.ShapeDtypeStruct((npad, d), jnp.float32),
    )


def _tc_pool_fc(npad, d, c):
    grid = npad // _BLK

    def body(h_ref, batch_ref, wfc_ref, bfc_ref,
             out_ref, pooled_ref, counts_ref):
        i = pl.program_id(0)

        @pl.when(i == 0)
        def _():
            pooled_ref[...] = jnp.zeros_like(pooled_ref)
            counts_ref[...] = jnp.zeros_like(counts_ref)
            out_ref[...] = jnp.zeros_like(out_ref)

        bvals = batch_ref[0]
        seg = lax.broadcasted_iota(jnp.int32, (_NSEG, _BLK), 0)
        onehot = jnp.where(seg == bvals, 1.0, 0.0)
        pooled_ref[...] += jnp.dot(onehot, h_ref[...],
                                   preferred_element_type=jnp.float32)
        counts_ref[...] += jnp.sum(onehot, axis=1, keepdims=True)

        @pl.when(i == grid - 1)
        def _():
            pooled = pooled_ref[...] / jnp.maximum(counts_ref[...], 1.0)
            out_ref[...] = (jnp.dot(pooled, wfc_ref[...],
                                    preferred_element_type=jnp.float32)
                            + bfc_ref[...])

    return pl.pallas_call(
        body,
        grid=(grid,),
        in_specs=[
            pl.BlockSpec((_BLK, d), lambda i: (i, 0)),
            pl.BlockSpec((1, 1, _BLK), lambda i: (i, 0, 0)),
            pl.BlockSpec((d, c), lambda i: (0, 0)),
            pl.BlockSpec((1, c), lambda i: (0, 0)),
        ],
        out_specs=[
            pl.BlockSpec((_NSEG, c), lambda i: (0, 0)),
            pl.BlockSpec((_NSEG, d), lambda i: (0, 0)),
            pl.BlockSpec((_NSEG, 1), lambda i: (0, 0)),
        ],
        out_shape=[
            jax.ShapeDtypeStruct((_NSEG, c), jnp.float32),
            jax.ShapeDtypeStruct((_NSEG, d), jnp.float32),
            jax.ShapeDtypeStruct((_NSEG, 1), jnp.float32),
        ],
    )


def kernel(x, edge_index, batch, W1, b1, W2, b2, Wfc, bfc):
    n, d = x.shape
    e = edge_index.shape[1]
    c = Wfc.shape[1]
    npad = n + _PAD_ROWS

    cw = -(-e // (_NW * _CHUNK))
    cw += cw % 2
    epad = _NW * cw * _CHUNK
    dummy = n + (jnp.arange(epad - e, dtype=jnp.int32) % _PAD_ROWS)
    srcp = jnp.concatenate([edge_index[0], dummy]).reshape(_NW, cw, _CHUNK)
    dstp = jnp.concatenate([edge_index[1], dummy]).reshape(_NW, cw, _CHUNK)
    xp = jnp.concatenate([x, jnp.zeros((_PAD_ROWS, d), x.dtype)])

    deg_k = _build_degree_kernel(npad, cw)
    prop_k = _build_propagate_kernel(npad, d, cw)

    degp = deg_k(dstp).reshape(_NC, npad)

    hp1 = _tc_scale_in(npad, d)(xp, W1, degp)
    s1 = prop_k(hp1, srcp, dstp).reshape(_NC, npad, d)
    hp2 = _tc_layer_mid(npad, d, n)(s1, hp1, degp, b1.reshape(1, d), W2)
    s2 = prop_k(hp2, srcp, dstp).reshape(_NC, npad, d)
    h2 = _tc_layer_out(npad, d)(s2, hp2, degp, b2.reshape(1, d))

    batchp = jnp.concatenate(
        [batch, jnp.full((_PAD_ROWS,), _NSEG, batch.dtype)]
    ).reshape(npad // _BLK, 1, _BLK)
    out, _, _ = _tc_pool_fc(npad, d, c)(h2, batchp, Wfc, bfc.reshape(1, c))
    return out


# speedup vs baseline: 26.5846x; 26.5846x over previous
import functools

import jax
import jax.numpy as jnp
from jax import lax
from jax.experimental import pallas as pl
from jax.experimental.pallas import tpu as pltpu
from jax.experimental.pallas import tpu_sc as plsc

_NC = 2
_NS = 16
_NW = _NC * _NS

_CHUNK = 128
_PAD_ROWS = 240
_BLK = 512
_NSEG = 64


def _build_degree_kernel(npad, cw):
    rows_per_tile = npad // _NS
    mesh = plsc.VectorSubcoreMesh(core_axis_name="c", subcore_axis_name="s")

    @functools.partial(
        pl.kernel,
        out_type=jax.ShapeDtypeStruct((_NC, _NS, rows_per_tile), jnp.float32),
        mesh=mesh,
        scratch_types=[
            pltpu.VMEM((cw, _CHUNK), jnp.int32),
            pltpu.VMEM((_CHUNK,), jnp.float32),
            pltpu.VMEM((rows_per_tile,), jnp.float32),
            pltpu.VMEM_SHARED((npad,), jnp.float32),
        ],
    )
    def deg_kernel(dst_hbm, out_hbm, idx_v, ones_v, zero_v, acc_sh):
        c = lax.axis_index("c")
        s = lax.axis_index("s")
        w = s * _NC + c
        row0 = s * rows_per_tile

        def _fill(i, carry):
            zero_v[pl.ds(i * 16, 16)] = jnp.zeros((16,), jnp.float32)
            return carry

        lax.fori_loop(0, rows_per_tile // 16, _fill, 0)

        def _fill1(i, carry):
            ones_v[pl.ds(i * 16, 16)] = jnp.ones((16,), jnp.float32)
            return carry

        lax.fori_loop(0, _CHUNK // 16, _fill1, 0)

        pltpu.sync_copy(zero_v, acc_sh.at[pl.ds(row0, rows_per_tile)])
        pltpu.sync_copy(dst_hbm.at[w], idx_v)
        plsc.subcore_barrier()

        def _scat(j, carry):
            pltpu.sync_copy(ones_v, acc_sh.at[idx_v.at[j]], add=True)
            return carry

        lax.fori_loop(0, cw, _scat, 0)
        plsc.subcore_barrier()
        pltpu.sync_copy(acc_sh.at[pl.ds(row0, rows_per_tile)], out_hbm.at[c, s])

    return deg_kernel


_SEG = 16


def _build_propagate_kernel(npad, d, cw):
    rows_per_tile = npad // _NS
    nzero = rows_per_tile // _CHUNK
    nseg = cw // _SEG
    mesh = plsc.VectorSubcoreMesh(core_axis_name="c", subcore_axis_name="s")

    @functools.partial(
        pl.kernel,
        out_type=jax.ShapeDtypeStruct((_NC, _NS, rows_per_tile, d), jnp.float32),
        mesh=mesh,
        scratch_types=[
            pltpu.VMEM((_SEG, _CHUNK), jnp.int32),
            pltpu.VMEM((_SEG, _CHUNK), jnp.int32),
            pltpu.VMEM((2, _CHUNK, d), jnp.float32),
            pltpu.VMEM_SHARED((npad, d), jnp.float32),
            pltpu.SemaphoreType.DMA,
            pltpu.SemaphoreType.DMA,
        ],
    )
    def prop_kernel(h_hbm, src_hbm, dst_hbm, out_hbm,
                    sidx_v, didx_v, rows_v, acc_sh, sem0, sem1):
        c = lax.axis_index("c")
        s = lax.axis_index("s")
        w = s * _NC + c
        row0 = s * rows_per_tile

        def _zr(r, carry):
            for kk in range(d // 16):
                rows_v[0, r, pl.ds(kk * 16, 16)] = jnp.zeros((16,), jnp.float32)
            return carry

        lax.fori_loop(0, _CHUNK, _zr, 0)
        for k in range(nzero):
            pltpu.sync_copy(rows_v.at[0],
                            acc_sh.at[pl.ds(row0 + k * _CHUNK, _CHUNK)])
        plsc.subcore_barrier()

        def _seg(g, carry):
            pltpu.sync_copy(src_hbm.at[w, pl.ds(g * _SEG, _SEG)], sidx_v)
            pltpu.sync_copy(dst_hbm.at[w, pl.ds(g * _SEG, _SEG)], didx_v)
            pltpu.async_copy(h_hbm.at[sidx_v.at[0]], rows_v.at[0], sem0)

            def _pair(p, carry2):
                j0 = 2 * p
                pltpu.make_async_copy(h_hbm.at[sidx_v.at[j0]],
                                      rows_v.at[0], sem0).wait()
                pltpu.async_copy(h_hbm.at[sidx_v.at[j0 + 1]], rows_v.at[1],
                                 sem1)
                pltpu.sync_copy(rows_v.at[0], acc_sh.at[didx_v.at[j0]],
                                add=True)
                pltpu.make_async_copy(h_hbm.at[sidx_v.at[j0 + 1]],
                                      rows_v.at[1], sem1).wait()

                @pl.when(j0 + 2 < _SEG)
                def _():
                    pltpu.async_copy(h_hbm.at[sidx_v.at[j0 + 2]],
                                     rows_v.at[0], sem0)

                pltpu.sync_copy(rows_v.at[1], acc_sh.at[didx_v.at[j0 + 1]],
                                add=True)
                return carry2

            lax.fori_loop(0, _SEG // 2, _pair, 0)
            return carry

        lax.fori_loop(0, nseg, _seg, 0)
        plsc.subcore_barrier()
        pltpu.sync_copy(acc_sh.at[pl.ds(row0, rows_per_tile)],
                        out_hbm.at[c, s])

    return prop_kernel


def _tc_scale_in(npad, d):
    grid = npad // _BLK

    def body(x_ref, w_ref, degp_ref, out_ref):
        deg = degp_ref[0] + degp_ref[1] + 1.0
        dinv = lax.rsqrt(deg)
        h = jnp.dot(x_ref[...], w_ref[...], preferred_element_type=jnp.float32)
        out_ref[...] = h * dinv[:, None]

    return pl.pallas_call(
        body,
        grid=(grid,),
        in_specs=[
            pl.BlockSpec((_BLK, d), lambda i: (i, 0)),
            pl.BlockSpec((d, d), lambda i: (0, 0)),
            pl.BlockSpec((_NC, _BLK), lambda i: (0, i)),
        ],
        out_specs=pl.BlockSpec((_BLK, d), lambda i: (i, 0)),
        out_shape=jax.ShapeDtypeStruct((npad, d), jnp.float32),
    )


def _tc_layer_mid(npad, d, n):
    grid = npad // _BLK

    def body(s_ref, hp_ref, degp_ref, b_ref, w_ref, out_ref):
        i = pl.program_id(0)
        deg = degp_ref[0] + degp_ref[1] + 1.0
        dinv = lax.rsqrt(deg)[:, None]
        tot = s_ref[0] + s_ref[1] + hp_ref[...]
        h1 = jnp.maximum(tot * dinv + b_ref[...], 0.0)
        out = jnp.dot(h1, w_ref[...], preferred_element_type=jnp.float32) * dinv
        rid = i * _BLK + lax.broadcasted_iota(jnp.int32, (_BLK, 1), 0)
        out_ref[...] = jnp.where(rid < n, out, 0.0)

    return pl.pallas_call(
        body,
        grid=(grid,),
        in_specs=[
            pl.BlockSpec((_NC, _BLK, d), lambda i: (0, i, 0)),
            pl.BlockSpec((_BLK, d), lambda i: (i, 0)),
            pl.BlockSpec((_NC, _BLK), lambda i: (0, i)),
            pl.BlockSpec((1, d), lambda i: (0, 0)),
            pl.BlockSpec((d, d), lambda i: (0, 0)),
        ],
        out_specs=pl.BlockSpec((_BLK, d), lambda i: (i, 0)),
        out_shape=jax.ShapeDtypeStruct((npad, d), jnp.float32),
    )


def _tc_layer_out(npad, d):
    grid = npad // _BLK

    def body(s_ref, hp_ref, degp_ref, b_ref, out_ref):
        deg = degp_ref[0] + degp_ref[1] + 1.0
        dinv = lax.rsqrt(deg)[:, None]
        tot = s_ref[0] + s_ref[1] + hp_ref[...]
        out_ref[...] = tot * dinv + b_ref[...]

    return pl.pallas_call(
        body,
        grid=(grid,),
        in_specs=[
            pl.BlockSpec((_NC, _BLK, d), lambda i: (0, i, 0)),
            pl.BlockSpec((_BLK, d), lambda i: (i, 0)),
            pl.BlockSpec((_NC, _BLK), lambda i: (0, i)),
            pl.BlockSpec((1, d), lambda i: (0, 0)),
        ],
        out_specs=pl.BlockSpec((_BLK, d), lambda i: (i, 0)),
        out_shape=jax.ShapeDtypeStruct((npad, d), jnp.float32),
    )


def _tc_pool_fc(npad, d, c):
    grid = npad // _BLK

    def body(h_ref, batch_ref, wfc_ref, bfc_ref,
             out_ref, pooled_ref, counts_ref):
        i = pl.program_id(0)

        @pl.when(i == 0)
        def _():
            pooled_ref[...] = jnp.zeros_like(pooled_ref)
            counts_ref[...] = jnp.zeros_like(counts_ref)
            out_ref[...] = jnp.zeros_like(out_ref)

        bvals = batch_ref[0]
        seg = lax.broadcasted_iota(jnp.int32, (_NSEG, _BLK), 0)
        onehot = jnp.where(seg == bvals, 1.0, 0.0)
        pooled_ref[...] += jnp.dot(onehot, h_ref[...],
                                   preferred_element_type=jnp.float32)
        counts_ref[...] += jnp.sum(onehot, axis=1, keepdims=True)

        @pl.when(i == grid - 1)
        def _():
            pooled = pooled_ref[...] / jnp.maximum(counts_ref[...], 1.0)
            out_ref[...] = (jnp.dot(pooled, wfc_ref[...],
                                    preferred_element_type=jnp.float32)
                            + bfc_ref[...])

    return pl.pallas_call(
        body,
        grid=(grid,),
        in_specs=[
            pl.BlockSpec((_BLK, d), lambda i: (i, 0)),
            pl.BlockSpec((1, 1, _BLK), lambda i: (i, 0, 0)),
            pl.BlockSpec((d, c), lambda i: (0, 0)),
            pl.BlockSpec((1, c), lambda i: (0, 0)),
        ],
        out_specs=[
            pl.BlockSpec((_NSEG, c), lambda i: (0, 0)),
            pl.BlockSpec((_NSEG, d), lambda i: (0, 0)),
            pl.BlockSpec((_NSEG, 1), lambda i: (0, 0)),
        ],
        out_shape=[
            jax.ShapeDtypeStruct((_NSEG, c), jnp.float32),
            jax.ShapeDtypeStruct((_NSEG, d), jnp.float32),
            jax.ShapeDtypeStruct((_NSEG, 1), jnp.float32),
        ],
    )


def kernel(x, edge_index, batch, W1, b1, W2, b2, Wfc, bfc):
    n, d = x.shape
    e = edge_index.shape[1]
    c = Wfc.shape[1]
    npad = n + _PAD_ROWS

    cw = -(-e // (_NW * _CHUNK))
    cw = -(-cw // _SEG) * _SEG
    epad = _NW * cw * _CHUNK
    dummy = n + (jnp.arange(epad - e, dtype=jnp.int32) % _PAD_ROWS)
    srcp = jnp.concatenate([edge_index[0], dummy]).reshape(_NW, cw, _CHUNK)
    dstp = jnp.concatenate([edge_index[1], dummy]).reshape(_NW, cw, _CHUNK)
    xp = jnp.concatenate([x, jnp.zeros((_PAD_ROWS, d), x.dtype)])

    deg_k = _build_degree_kernel(npad, cw)
    prop_k = _build_propagate_kernel(npad, d, cw)

    degp = deg_k(dstp).reshape(_NC, npad)

    hp1 = _tc_scale_in(npad, d)(xp, W1, degp)
    s1 = prop_k(hp1, srcp, dstp).reshape(_NC, npad, d)
    hp2 = _tc_layer_mid(npad, d, n)(s1, hp1, degp, b1.reshape(1, d), W2)
    s2 = prop_k(hp2, srcp, dstp).reshape(_NC, npad, d)
    h2 = _tc_layer_out(npad, d)(s2, hp2, degp, b2.reshape(1, d))

    batchp = jnp.concatenate(
        [batch, jnp.full((_PAD_ROWS,), _NSEG, batch.dtype)]
    ).reshape(npad // _BLK, 1, _BLK)
    out, _, _ = _tc_pool_fc(npad, d, c)(h2, batchp, Wfc, bfc.reshape(1, c))
    return out
